# baseline (speedup 1.0000x reference)
import functools

import jax
import jax.numpy as jnp
from jax import lax
from jax.experimental import pallas as pl
from jax.experimental.pallas import tpu as pltpu
from jax.experimental.pallas import tpu_sc as plsc

N = 10000
E = 320000
NF = 128
EF = 32
L = 3

NC = 2
NS = 16
NW = NC * NS
EPW = E // NW
CH = 80
NCHUNK = EPW // CH
NPAIR = (NCHUNK - 1) // 2
RPS = 632
N_PAD = NS * RPS

BE = 6400
NB = E // BE


def _mesh():
    return plsc.VectorSubcoreMesh(
        core_axis_name="c", subcore_axis_name="s",
        num_cores=NC, num_subcores=NS)


@functools.cache
def _gather_kernel():
    @functools.partial(
        pl.kernel,
        out_type=(jax.ShapeDtypeStruct((E, NF), jnp.float32),
                  jax.ShapeDtypeStruct((E, NF), jnp.float32)),
        mesh=_mesh(),
        scratch_types=(pltpu.VMEM((NCHUNK, CH), jnp.int32),
                       pltpu.VMEM((NCHUNK, CH), jnp.int32),
                       pltpu.VMEM((CH, NF), jnp.float32),
                       pltpu.VMEM((CH, NF), jnp.float32),
                       pltpu.VMEM((CH, NF), jnp.float32),
                       pltpu.VMEM((CH, NF), jnp.float32),
                       pltpu.SemaphoreType.DMA,
                       pltpu.SemaphoreType.DMA,
                       pltpu.SemaphoreType.DMA,
                       pltpu.SemaphoreType.DMA),
    )
    def k(h_hbm, src3_hbm, dst3_hbm, hs_hbm, hd_hbm,
          sidx, didx, rs0, rd0, rs1, rd1, ss0, sd0, ss1, sd1):
        wid = lax.axis_index("s") * NC + lax.axis_index("c")
        pltpu.sync_copy(src3_hbm.at[wid], sidx)
        pltpu.sync_copy(dst3_hbm.at[wid], didx)

        def start(c, bs, bd, sems_, semd_):
            pltpu.async_copy(h_hbm.at[sidx.at[c]], bs, sems_)
            pltpu.async_copy(h_hbm.at[didx.at[c]], bd, semd_)

        def wait(bs, bd, sems_, semd_):
            pltpu.make_async_copy(h_hbm.at[sidx.at[0]], bs, sems_).wait()
            pltpu.make_async_copy(h_hbm.at[didx.at[0]], bd, semd_).wait()

        def drain(c, bs, bd):
            base = pl.multiple_of(wid * EPW + c * CH, 8)
            pltpu.sync_copy(bs, hs_hbm.at[pl.ds(base, CH)])
            pltpu.sync_copy(bd, hd_hbm.at[pl.ds(base, CH)])

        start(0, rs0, rd0, ss0, sd0)

        @pl.loop(0, NPAIR)
        def _pair(t):
            c0 = 2 * t
            start(c0 + 1, rs1, rd1, ss1, sd1)
            wait(rs0, rd0, ss0, sd0)
            drain(c0, rs0, rd0)
            start(c0 + 2, rs0, rd0, ss0, sd0)
            wait(rs1, rd1, ss1, sd1)
            drain(c0 + 1, rs1, rd1)

        wait(rs0, rd0, ss0, sd0)
        drain(NCHUNK - 1, rs0, rd0)

    return k


def _sc_gather(h, src3, dst3):
    return _gather_kernel()(h, src3, dst3)


@functools.cache
def _scatter_kernel(width):
    @functools.partial(
        pl.kernel,
        out_type=jax.ShapeDtypeStruct((NC, N_PAD, width), jnp.float32),
        mesh=_mesh(),
        scratch_types=(pltpu.VMEM_SHARED((N_PAD, width), jnp.float32),
                       pltpu.VMEM((CH, width), jnp.float32),
                       pltpu.VMEM((CH, width), jnp.float32),
                       pltpu.VMEM((NCHUNK, CH), jnp.int32),
                       pltpu.SemaphoreType.DMA,
                       pltpu.SemaphoreType.DMA),
    )
    def k(val_hbm, idx3_hbm, zeros_hbm, out_hbm, acc_sh, buf0, buf1, didx,
          sm0, sm1):
        cid = lax.axis_index("c")
        sid = lax.axis_index("s")
        wid = sid * NC + cid
        row0 = pl.multiple_of(sid * RPS, 8)
        pltpu.sync_copy(zeros_hbm.at[pl.ds(row0, RPS)],
                        acc_sh.at[pl.ds(row0, RPS)])
        pltpu.sync_copy(idx3_hbm.at[wid], didx)
        plsc.subcore_barrier()

        def start(c, buf, sem):
            base = pl.multiple_of(wid * EPW + c * CH, 8)
            pltpu.async_copy(val_hbm.at[pl.ds(base, CH)], buf, sem)

        def wait(buf, sem):
            pltpu.make_async_copy(val_hbm.at[pl.ds(0, CH)], buf, sem).wait()

        def scat(c, buf):
            pltpu.sync_copy(buf, acc_sh.at[didx.at[c]], add=True)

        start(0, buf0, sm0)

        @pl.loop(0, NPAIR)
        def _pair(t):
            c0 = 2 * t
            start(c0 + 1, buf1, sm1)
            wait(buf0, sm0)
            scat(c0, buf0)
            start(c0 + 2, buf0, sm0)
            wait(buf1, sm1)
            scat(c0 + 1, buf1)

        wait(buf0, sm0)
        scat(NCHUNK - 1, buf0)

        plsc.subcore_barrier()
        pltpu.sync_copy(acc_sh.at[pl.ds(row0, RPS)],
                        out_hbm.at[cid, pl.ds(row0, RPS)])

    return k


def _sc_scatter(vals, idx3, zeros):
    return _scatter_kernel(vals.shape[1])(vals, idx3, zeros)


def _rbf(bond_blk):
    centers = 1.0 + (4.0 / (EF - 1)) * lax.broadcasted_iota(
        jnp.int32, (1, EF), 1).astype(jnp.float32)
    gamma = (EF - 1) / 4.0
    return jnp.exp(-gamma * (bond_blk - centers) ** 2)


def _edge_x(hs, hd, bond, ws, wd, we, bcat):
    e = _rbf(bond)
    bf = jnp.bfloat16
    x = jnp.dot(hs.astype(bf), ws.astype(bf),
                preferred_element_type=jnp.float32)
    x += jnp.dot(hd.astype(bf), wd.astype(bf),
                 preferred_element_type=jnp.float32)
    x += jnp.dot(e.astype(bf), we.astype(bf),
                 preferred_element_type=jnp.float32)
    return x + bcat


def _edge_body(hs_ref, hd_ref, bond_ref, ws_ref, wd_ref, we_ref, bcat_ref,
               g_ref, bt_ref, out_ref, acc_ref, sc_ref):
    i = pl.program_id(0)

    @pl.when(i == 0)
    def _():
        acc_ref[...] = jnp.zeros_like(acc_ref)

    @pl.when(i < NB)
    def _stats():
        x = _edge_x(hs_ref[...], hd_ref[...], bond_ref[...],
                    ws_ref[...], wd_ref[...], we_ref[...], bcat_ref[...])
        acc_ref[0:1, :] += jnp.sum(x, axis=0, keepdims=True)
        acc_ref[1:2, :] += jnp.sum(x * x, axis=0, keepdims=True)

    @pl.when(i >= NB)
    def _msg():
        @pl.when(i == NB)
        def _fin():
            mean = acc_ref[0:1, :] * (1.0 / E)
            var = acc_ref[1:2, :] * (1.0 / E) - mean * mean
            scale = g_ref[...] * lax.rsqrt(var + 1e-5)
            sc_ref[0:1, :] = scale
            sc_ref[1:2, :] = bt_ref[...] - mean * scale

        x = _edge_x(hs_ref[...], hd_ref[...], bond_ref[...],
                    ws_ref[...], wd_ref[...], we_ref[...], bcat_ref[...])
        xh = x * sc_ref[0:1, :] + sc_ref[1:2, :]
        xi = xh[:, :NF]
        xu = xh[:, NF:]
        sig = 1.0 / (1.0 + jnp.exp(-xi))
        sp = jnp.maximum(xu, 0.0) + jnp.log(1.0 + jnp.exp(-jnp.abs(xu)))
        out_ref[...] = sig * sp


def _edge_fused(hs, hd, bond, ws, wd, we, bcat, gcat, btcat):
    blk = lambda i: (lax.rem(i, NB), 0)
    cst = lambda i: (0, 0)
    return pl.pallas_call(
        _edge_body,
        grid=(2 * NB,),
        in_specs=[
            pl.BlockSpec((BE, NF), blk),
            pl.BlockSpec((BE, NF), blk),
            pl.BlockSpec((BE, 1), blk),
            pl.BlockSpec((NF, 2 * NF), cst),
            pl.BlockSpec((NF, 2 * NF), cst),
            pl.BlockSpec((EF, 2 * NF), cst),
            pl.BlockSpec((1, 2 * NF), cst),
            pl.BlockSpec((1, 2 * NF), cst),
            pl.BlockSpec((1, 2 * NF), cst),
        ],
        out_specs=pl.BlockSpec((BE, NF), blk),
        out_shape=jax.ShapeDtypeStruct((E, NF), jnp.float32),
        scratch_shapes=[pltpu.VMEM((8, 2 * NF), jnp.float32),
                        pltpu.VMEM((8, 2 * NF), jnp.float32)],
    )(hs, hd, bond, ws, wd, we, bcat, gcat, btcat)


def _node_body(a0_ref, a1_ref, h_ref, g_ref, b_ref, out_ref):
    agg = a0_ref[0] + a1_ref[0]
    m = jnp.mean(agg, axis=0, keepdims=True)
    d = agg - m
    v = jnp.mean(d * d, axis=0, keepdims=True)
    bn = g_ref[...] * d * lax.rsqrt(v + 1e-5) + b_ref[...]
    y = h_ref[...] + bn
    out_ref[...] = jnp.maximum(y, 0.0) + jnp.log(1.0 + jnp.exp(-jnp.abs(y)))


def _node_update(aggp, h, g, b):
    return pl.pallas_call(
        _node_body,
        grid=(1,),
        in_specs=[
            pl.BlockSpec((1, N, NF), lambda i: (0, 0, 0)),
            pl.BlockSpec((1, N, NF), lambda i: (1, 0, 0)),
            pl.BlockSpec((N, NF), lambda i: (0, 0)),
            pl.BlockSpec((1, NF), lambda i: (0, 0)),
            pl.BlockSpec((1, NF), lambda i: (0, 0)),
        ],
        out_specs=pl.BlockSpec((N, NF), lambda i: (0, 0)),
        out_shape=jax.ShapeDtypeStruct((N, NF), jnp.float32),
    )(aggp, aggp, h, g, b)


def kernel(atom_features, bondlength, edge_index, W_emb, b_emb, Wi, bi, gi,
           bti, Wu, bu, gu, btu, gbn, bbn, W_fc, b_fc, W_out, b_out):
    src3 = edge_index[0].astype(jnp.int32).reshape(NW, NCHUNK, CH)
    dst3 = edge_index[1].astype(jnp.int32).reshape(NW, NCHUNK, CH)
    bond = bondlength.reshape(E, 1)
    zeros_n = jnp.zeros((N_PAD, NF), jnp.float32)

    h = atom_features @ W_emb + b_emb

    for l in range(L):
        ws = jnp.concatenate([Wi[l][:NF], Wu[l][:NF]], axis=1)
        wd = jnp.concatenate([Wi[l][NF:2 * NF], Wu[l][NF:2 * NF]], axis=1)
        we = jnp.concatenate([Wi[l][2 * NF:], Wu[l][2 * NF:]], axis=1)
        bcat = jnp.concatenate([bi[l], bu[l]]).reshape(1, 2 * NF)
        gcat = jnp.concatenate([gi[l], gu[l]]).reshape(1, 2 * NF)
        btcat = jnp.concatenate([bti[l], btu[l]]).reshape(1, 2 * NF)

        hs, hd = _sc_gather(h, src3, dst3)
        msg = _edge_fused(hs, hd, bond, ws, wd, we, bcat, gcat, btcat)
        aggp = _sc_scatter(msg, dst3, zeros_n)
        h = _node_update(aggp, h, gbn[l].reshape(1, NF), bbn[l].reshape(1, NF))

    feat = jax.nn.softplus(h.mean(axis=0))
    feat = jax.nn.softplus(feat @ W_fc + b_fc)
    out = feat @ W_out + b_out
    return jnp.squeeze(out)


# speedup vs baseline: 1.0402x; 1.0402x over previous
import functools

import jax
import jax.numpy as jnp
from jax import lax
from jax.experimental import pallas as pl
from jax.experimental.pallas import tpu as pltpu
from jax.experimental.pallas import tpu_sc as plsc

N = 10000
E = 320000
NF = 128
EF = 32
L = 3

NC = 2
NS = 16
NW = NC * NS
EPW = E // NW
CH = 80
NCHUNK = EPW // CH
NPAIR = (NCHUNK - 1) // 2
RPS = 632
N_PAD = NS * RPS

BE = 6400
NB = E // BE


def _mesh():
    return plsc.VectorSubcoreMesh(
        core_axis_name="c", subcore_axis_name="s",
        num_cores=NC, num_subcores=NS)


@functools.cache
def _gather_kernel():
    @functools.partial(
        pl.kernel,
        out_type=(jax.ShapeDtypeStruct((E, NF), jnp.float32),
                  jax.ShapeDtypeStruct((E, NF), jnp.float32)),
        mesh=_mesh(),
        scratch_types=(pltpu.VMEM((NCHUNK, CH), jnp.int32),
                       pltpu.VMEM((NCHUNK, CH), jnp.int32),
                       pltpu.VMEM((CH, NF), jnp.float32),
                       pltpu.VMEM((CH, NF), jnp.float32),
                       pltpu.VMEM((CH, NF), jnp.float32),
                       pltpu.VMEM((CH, NF), jnp.float32),
                       pltpu.SemaphoreType.DMA,
                       pltpu.SemaphoreType.DMA,
                       pltpu.SemaphoreType.DMA,
                       pltpu.SemaphoreType.DMA),
    )
    def k(h_hbm, src3_hbm, dst3_hbm, hs_hbm, hd_hbm,
          sidx, didx, rs0, rd0, rs1, rd1, ss0, sd0, ss1, sd1):
        wid = lax.axis_index("s") * NC + lax.axis_index("c")
        pltpu.sync_copy(src3_hbm.at[wid], sidx)
        pltpu.sync_copy(dst3_hbm.at[wid], didx)

        def start(c, bs, bd, sems_, semd_):
            pltpu.async_copy(h_hbm.at[sidx.at[c]], bs, sems_)
            pltpu.async_copy(h_hbm.at[didx.at[c]], bd, semd_)

        def wait(bs, bd, sems_, semd_):
            pltpu.make_async_copy(h_hbm.at[sidx.at[0]], bs, sems_).wait()
            pltpu.make_async_copy(h_hbm.at[didx.at[0]], bd, semd_).wait()

        def drain(c, bs, bd):
            base = pl.multiple_of(wid * EPW + c * CH, 8)
            pltpu.sync_copy(bs, hs_hbm.at[pl.ds(base, CH)])
            pltpu.sync_copy(bd, hd_hbm.at[pl.ds(base, CH)])

        start(0, rs0, rd0, ss0, sd0)

        @pl.loop(0, NPAIR)
        def _pair(t):
            c0 = 2 * t
            start(c0 + 1, rs1, rd1, ss1, sd1)
            wait(rs0, rd0, ss0, sd0)
            drain(c0, rs0, rd0)
            start(c0 + 2, rs0, rd0, ss0, sd0)
            wait(rs1, rd1, ss1, sd1)
            drain(c0 + 1, rs1, rd1)

        wait(rs0, rd0, ss0, sd0)
        drain(NCHUNK - 1, rs0, rd0)

    return k


def _sc_gather(h, src3, dst3):
    return _gather_kernel()(h, src3, dst3)


@functools.cache
def _scatter_kernel(width):
    @functools.partial(
        pl.kernel,
        out_type=jax.ShapeDtypeStruct((NC, N_PAD, width), jnp.float32),
        mesh=_mesh(),
        scratch_types=(pltpu.VMEM_SHARED((N_PAD, width), jnp.float32),
                       pltpu.VMEM((CH, width), jnp.float32),
                       pltpu.VMEM((CH, width), jnp.float32),
                       pltpu.VMEM((NCHUNK, CH), jnp.int32),
                       pltpu.SemaphoreType.DMA,
                       pltpu.SemaphoreType.DMA),
    )
    def k(val_hbm, idx3_hbm, zeros_hbm, out_hbm, acc_sh, buf0, buf1, didx,
          sm0, sm1):
        cid = lax.axis_index("c")
        sid = lax.axis_index("s")
        wid = sid * NC + cid
        row0 = pl.multiple_of(sid * RPS, 8)
        pltpu.sync_copy(zeros_hbm.at[pl.ds(row0, RPS)],
                        acc_sh.at[pl.ds(row0, RPS)])
        pltpu.sync_copy(idx3_hbm.at[wid], didx)
        plsc.subcore_barrier()

        def start(c, buf, sem):
            base = pl.multiple_of(wid * EPW + c * CH, 8)
            pltpu.async_copy(val_hbm.at[pl.ds(base, CH)], buf, sem)

        def wait(buf, sem):
            pltpu.make_async_copy(val_hbm.at[pl.ds(0, CH)], buf, sem).wait()

        def scat(c, buf):
            pltpu.sync_copy(buf, acc_sh.at[didx.at[c]], add=True)

        start(0, buf0, sm0)

        @pl.loop(0, NPAIR)
        def _pair(t):
            c0 = 2 * t
            start(c0 + 1, buf1, sm1)
            wait(buf0, sm0)
            scat(c0, buf0)
            start(c0 + 2, buf0, sm0)
            wait(buf1, sm1)
            scat(c0 + 1, buf1)

        wait(buf0, sm0)
        scat(NCHUNK - 1, buf0)

        plsc.subcore_barrier()
        pltpu.sync_copy(acc_sh.at[pl.ds(row0, RPS)],
                        out_hbm.at[cid, pl.ds(row0, RPS)])

    return k


def _sc_scatter(vals, idx3, zeros):
    return _scatter_kernel(vals.shape[1])(vals, idx3, zeros)


def _rbf(bond_blk):
    centers = 1.0 + (4.0 / (EF - 1)) * lax.broadcasted_iota(
        jnp.int32, (1, EF), 1).astype(jnp.float32)
    gamma = (EF - 1) / 4.0
    return jnp.exp(-gamma * (bond_blk - centers) ** 2)


def _edge_x(hs, hd, bond, ws, wd, we, bcat):
    e = _rbf(bond)
    bf = jnp.bfloat16
    x = jnp.dot(hs.astype(bf), ws.astype(bf),
                preferred_element_type=jnp.float32)
    x += jnp.dot(hd.astype(bf), wd.astype(bf),
                 preferred_element_type=jnp.float32)
    x += jnp.dot(e.astype(bf), we.astype(bf),
                 preferred_element_type=jnp.float32)
    return x + bcat


def _stats_body(hs_ref, hd_ref, bond_ref, ws_ref, wd_ref, we_ref, bcat_ref,
                out_ref):
    i = pl.program_id(0)

    @pl.when(i == 0)
    def _():
        out_ref[...] = jnp.zeros_like(out_ref)

    x = _edge_x(hs_ref[...], hd_ref[...], bond_ref[...],
                ws_ref[...], wd_ref[...], we_ref[...], bcat_ref[...])
    out_ref[0:1, :] += jnp.sum(x, axis=0, keepdims=True)
    out_ref[1:2, :] += jnp.sum(x * x, axis=0, keepdims=True)


def _edge_stats(hs, hd, bond, ws, wd, we, bcat):
    return pl.pallas_call(
        _stats_body,
        grid=(NB,),
        in_specs=[
            pl.BlockSpec((BE, NF), lambda i: (i, 0)),
            pl.BlockSpec((BE, NF), lambda i: (i, 0)),
            pl.BlockSpec((BE, 1), lambda i: (i, 0)),
            pl.BlockSpec((NF, 2 * NF), lambda i: (0, 0)),
            pl.BlockSpec((NF, 2 * NF), lambda i: (0, 0)),
            pl.BlockSpec((EF, 2 * NF), lambda i: (0, 0)),
            pl.BlockSpec((1, 2 * NF), lambda i: (0, 0)),
        ],
        out_specs=pl.BlockSpec((8, 2 * NF), lambda i: (0, 0)),
        out_shape=jax.ShapeDtypeStruct((8, 2 * NF), jnp.float32),
    )(hs, hd, bond, ws, wd, we, bcat)


def _msg_body(hs_ref, hd_ref, bond_ref, ws_ref, wd_ref, we_ref, bcat_ref,
              scale_ref, shift_ref, out_ref):
    x = _edge_x(hs_ref[...], hd_ref[...], bond_ref[...],
                ws_ref[...], wd_ref[...], we_ref[...], bcat_ref[...])
    xh = x * scale_ref[...] + shift_ref[...]
    xi = xh[:, :NF]
    xu = xh[:, NF:]
    sig = 1.0 / (1.0 + jnp.exp(-xi))
    sp = jnp.maximum(xu, 0.0) + jnp.log(1.0 + jnp.exp(-jnp.abs(xu)))
    out_ref[...] = sig * sp


def _edge_msg(hs, hd, bond, ws, wd, we, bcat, scale, shift):
    return pl.pallas_call(
        _msg_body,
        grid=(NB,),
        in_specs=[
            pl.BlockSpec((BE, NF), lambda i: (i, 0)),
            pl.BlockSpec((BE, NF), lambda i: (i, 0)),
            pl.BlockSpec((BE, 1), lambda i: (i, 0)),
            pl.BlockSpec((NF, 2 * NF), lambda i: (0, 0)),
            pl.BlockSpec((NF, 2 * NF), lambda i: (0, 0)),
            pl.BlockSpec((EF, 2 * NF), lambda i: (0, 0)),
            pl.BlockSpec((1, 2 * NF), lambda i: (0, 0)),
            pl.BlockSpec((1, 2 * NF), lambda i: (0, 0)),
            pl.BlockSpec((1, 2 * NF), lambda i: (0, 0)),
        ],
        out_specs=pl.BlockSpec((BE, NF), lambda i: (i, 0)),
        out_shape=jax.ShapeDtypeStruct((E, NF), jnp.float32),
    )(hs, hd, bond, ws, wd, we, bcat, scale, shift)


def _node_body(a0_ref, a1_ref, h_ref, g_ref, b_ref, out_ref):
    agg = a0_ref[0] + a1_ref[0]
    m = jnp.mean(agg, axis=0, keepdims=True)
    d = agg - m
    v = jnp.mean(d * d, axis=0, keepdims=True)
    bn = g_ref[...] * d * lax.rsqrt(v + 1e-5) + b_ref[...]
    y = h_ref[...] + bn
    out_ref[...] = jnp.maximum(y, 0.0) + jnp.log(1.0 + jnp.exp(-jnp.abs(y)))


def _node_update(aggp, h, g, b):
    return pl.pallas_call(
        _node_body,
        grid=(1,),
        in_specs=[
            pl.BlockSpec((1, N, NF), lambda i: (0, 0, 0)),
            pl.BlockSpec((1, N, NF), lambda i: (1, 0, 0)),
            pl.BlockSpec((N, NF), lambda i: (0, 0)),
            pl.BlockSpec((1, NF), lambda i: (0, 0)),
            pl.BlockSpec((1, NF), lambda i: (0, 0)),
        ],
        out_specs=pl.BlockSpec((N, NF), lambda i: (0, 0)),
        out_shape=jax.ShapeDtypeStruct((N, NF), jnp.float32),
    )(aggp, aggp, h, g, b)


def kernel(atom_features, bondlength, edge_index, W_emb, b_emb, Wi, bi, gi,
           bti, Wu, bu, gu, btu, gbn, bbn, W_fc, b_fc, W_out, b_out):
    src3 = edge_index[0].astype(jnp.int32).reshape(NW, NCHUNK, CH)
    dst3 = edge_index[1].astype(jnp.int32).reshape(NW, NCHUNK, CH)
    bond = bondlength.reshape(E, 1)
    zeros_n = jnp.zeros((N_PAD, NF), jnp.float32)

    h = atom_features @ W_emb + b_emb

    for l in range(L):
        ws = jnp.concatenate([Wi[l][:NF], Wu[l][:NF]], axis=1)
        wd = jnp.concatenate([Wi[l][NF:2 * NF], Wu[l][NF:2 * NF]], axis=1)
        we = jnp.concatenate([Wi[l][2 * NF:], Wu[l][2 * NF:]], axis=1)
        bcat = jnp.concatenate([bi[l], bu[l]]).reshape(1, 2 * NF)
        gcat = jnp.concatenate([gi[l], gu[l]])
        btcat = jnp.concatenate([bti[l], btu[l]])

        hs, hd = _sc_gather(h, src3, dst3)
        sums = _edge_stats(hs, hd, bond, ws, wd, we, bcat)
        mean = sums[0] / E
        var = sums[1] / E - mean * mean
        scale = (gcat * lax.rsqrt(var + 1e-5)).reshape(1, 2 * NF)
        shift = (btcat - mean * scale[0]).reshape(1, 2 * NF)
        msg = _edge_msg(hs, hd, bond, ws, wd, we, bcat, scale, shift)
        aggp = _sc_scatter(msg, dst3, zeros_n)
        h = _node_update(aggp, h, gbn[l].reshape(1, NF), bbn[l].reshape(1, NF))

    feat = jax.nn.softplus(h.mean(axis=0))
    feat = jax.nn.softplus(feat @ W_fc + b_fc)
    out = feat @ W_out + b_out
    return jnp.squeeze(out)


# speedup vs baseline: 1.0546x; 1.0138x over previous
import functools

import jax
import jax.numpy as jnp
from jax import lax
from jax.experimental import pallas as pl
from jax.experimental.pallas import tpu as pltpu
from jax.experimental.pallas import tpu_sc as plsc

N = 10000
E = 320000
NF = 128
EF = 32
L = 3

NC = 2
NS = 16
NW = NC * NS
EPW = E // NW
CH = 80
NCHUNK = EPW // CH
NPAIR = (NCHUNK - 1) // 2
RPS = 632
N_PAD = NS * RPS

BE = 8000
NB = E // BE


def _mesh():
    return plsc.VectorSubcoreMesh(
        core_axis_name="c", subcore_axis_name="s",
        num_cores=NC, num_subcores=NS)


@functools.cache
def _gather_kernel():
    @functools.partial(
        pl.kernel,
        out_type=(jax.ShapeDtypeStruct((E, NF), jnp.float32),
                  jax.ShapeDtypeStruct((E, NF), jnp.float32)),
        mesh=_mesh(),
        scratch_types=(pltpu.VMEM((NCHUNK, CH), jnp.int32),
                       pltpu.VMEM((NCHUNK, CH), jnp.int32),
                       pltpu.VMEM((CH, NF), jnp.float32),
                       pltpu.VMEM((CH, NF), jnp.float32),
                       pltpu.VMEM((CH, NF), jnp.float32),
                       pltpu.VMEM((CH, NF), jnp.float32),
                       pltpu.SemaphoreType.DMA,
                       pltpu.SemaphoreType.DMA,
                       pltpu.SemaphoreType.DMA,
                       pltpu.SemaphoreType.DMA),
    )
    def k(h_hbm, src3_hbm, dst3_hbm, hs_hbm, hd_hbm,
          sidx, didx, rs0, rd0, rs1, rd1, ss0, sd0, ss1, sd1):
        wid = lax.axis_index("s") * NC + lax.axis_index("c")
        pltpu.sync_copy(src3_hbm.at[wid], sidx)
        pltpu.sync_copy(dst3_hbm.at[wid], didx)

        def start(c, bs, bd, sems_, semd_):
            pltpu.async_copy(h_hbm.at[sidx.at[c]], bs, sems_)
            pltpu.async_copy(h_hbm.at[didx.at[c]], bd, semd_)

        def wait(bs, bd, sems_, semd_):
            pltpu.make_async_copy(h_hbm.at[sidx.at[0]], bs, sems_).wait()
            pltpu.make_async_copy(h_hbm.at[didx.at[0]], bd, semd_).wait()

        def drain(c, bs, bd):
            base = pl.multiple_of(wid * EPW + c * CH, 8)
            pltpu.sync_copy(bs, hs_hbm.at[pl.ds(base, CH)])
            pltpu.sync_copy(bd, hd_hbm.at[pl.ds(base, CH)])

        start(0, rs0, rd0, ss0, sd0)

        @pl.loop(0, NPAIR)
        def _pair(t):
            c0 = 2 * t
            start(c0 + 1, rs1, rd1, ss1, sd1)
            wait(rs0, rd0, ss0, sd0)
            drain(c0, rs0, rd0)
            start(c0 + 2, rs0, rd0, ss0, sd0)
            wait(rs1, rd1, ss1, sd1)
            drain(c0 + 1, rs1, rd1)

        wait(rs0, rd0, ss0, sd0)
        drain(NCHUNK - 1, rs0, rd0)

    return k


def _sc_gather(h, src3, dst3):
    return _gather_kernel()(h, src3, dst3)


@functools.cache
def _scatter_kernel(width):
    @functools.partial(
        pl.kernel,
        out_type=jax.ShapeDtypeStruct((NC, N_PAD, width), jnp.float32),
        mesh=_mesh(),
        scratch_types=(pltpu.VMEM_SHARED((N_PAD, width), jnp.float32),
                       pltpu.VMEM((CH, width), jnp.float32),
                       pltpu.VMEM((CH, width), jnp.float32),
                       pltpu.VMEM((NCHUNK, CH), jnp.int32),
                       pltpu.SemaphoreType.DMA,
                       pltpu.SemaphoreType.DMA),
    )
    def k(val_hbm, idx3_hbm, zeros_hbm, out_hbm, acc_sh, buf0, buf1, didx,
          sm0, sm1):
        cid = lax.axis_index("c")
        sid = lax.axis_index("s")
        wid = sid * NC + cid
        row0 = pl.multiple_of(sid * RPS, 8)
        pltpu.sync_copy(zeros_hbm.at[pl.ds(row0, RPS)],
                        acc_sh.at[pl.ds(row0, RPS)])
        pltpu.sync_copy(idx3_hbm.at[wid], didx)
        plsc.subcore_barrier()

        def start(c, buf, sem):
            base = pl.multiple_of(wid * EPW + c * CH, 8)
            pltpu.async_copy(val_hbm.at[pl.ds(base, CH)], buf, sem)

        def wait(buf, sem):
            pltpu.make_async_copy(val_hbm.at[pl.ds(0, CH)], buf, sem).wait()

        def scat(c, buf):
            pltpu.sync_copy(buf, acc_sh.at[didx.at[c]], add=True)

        start(0, buf0, sm0)

        @pl.loop(0, NPAIR)
        def _pair(t):
            c0 = 2 * t
            start(c0 + 1, buf1, sm1)
            wait(buf0, sm0)
            scat(c0, buf0)
            start(c0 + 2, buf0, sm0)
            wait(buf1, sm1)
            scat(c0 + 1, buf1)

        wait(buf0, sm0)
        scat(NCHUNK - 1, buf0)

        plsc.subcore_barrier()
        pltpu.sync_copy(acc_sh.at[pl.ds(row0, RPS)],
                        out_hbm.at[cid, pl.ds(row0, RPS)])

    return k


def _sc_scatter(vals, idx3, zeros):
    return _scatter_kernel(vals.shape[1])(vals, idx3, zeros)


def _rbf(bond_blk):
    centers = 1.0 + (4.0 / (EF - 1)) * lax.broadcasted_iota(
        jnp.int32, (1, EF), 1).astype(jnp.float32)
    gamma = (EF - 1) / 4.0
    return jnp.exp(-gamma * (bond_blk - centers) ** 2)


def _edge_x(hs, hd, bond, ws, wd, we, bcat):
    e = _rbf(bond)
    bf = jnp.bfloat16
    x = jnp.dot(hs.astype(bf), ws.astype(bf),
                preferred_element_type=jnp.float32)
    x += jnp.dot(hd.astype(bf), wd.astype(bf),
                 preferred_element_type=jnp.float32)
    x += jnp.dot(e.astype(bf), we.astype(bf),
                 preferred_element_type=jnp.float32)
    return x + bcat


def _stats_body(hs_ref, hd_ref, bond_ref, ws_ref, wd_ref, we_ref, bcat_ref,
                out_ref):
    i = pl.program_id(0)

    @pl.when(i == 0)
    def _():
        out_ref[...] = jnp.zeros_like(out_ref)

    x = _edge_x(hs_ref[...], hd_ref[...], bond_ref[...],
                ws_ref[...], wd_ref[...], we_ref[...], bcat_ref[...])
    out_ref[0:1, :] += jnp.sum(x, axis=0, keepdims=True)
    out_ref[1:2, :] += jnp.sum(x * x, axis=0, keepdims=True)


def _edge_stats(hs, hd, bond, ws, wd, we, bcat):
    return pl.pallas_call(
        _stats_body,
        grid=(NB,),
        in_specs=[
            pl.BlockSpec((BE, NF), lambda i: (i, 0)),
            pl.BlockSpec((BE, NF), lambda i: (i, 0)),
            pl.BlockSpec((BE, 1), lambda i: (i, 0)),
            pl.BlockSpec((NF, 2 * NF), lambda i: (0, 0)),
            pl.BlockSpec((NF, 2 * NF), lambda i: (0, 0)),
            pl.BlockSpec((EF, 2 * NF), lambda i: (0, 0)),
            pl.BlockSpec((1, 2 * NF), lambda i: (0, 0)),
        ],
        out_specs=pl.BlockSpec((8, 2 * NF), lambda i: (0, 0)),
        out_shape=jax.ShapeDtypeStruct((8, 2 * NF), jnp.float32),
    )(hs, hd, bond, ws, wd, we, bcat)


def _msg_body(hs_ref, hd_ref, bond_ref, ws_ref, wd_ref, we_ref, bcat_ref,
              scale_ref, shift_ref, out_ref):
    x = _edge_x(hs_ref[...], hd_ref[...], bond_ref[...],
                ws_ref[...], wd_ref[...], we_ref[...], bcat_ref[...])
    xh = x * scale_ref[...] + shift_ref[...]
    xi = xh[:, :NF]
    xu = xh[:, NF:]
    sig = 1.0 / (1.0 + jnp.exp(-xi))
    sp = jnp.maximum(xu, 0.0) + jnp.log(1.0 + jnp.exp(-jnp.abs(xu)))
    out_ref[...] = sig * sp


def _edge_msg(hs, hd, bond, ws, wd, we, bcat, scale, shift):
    return pl.pallas_call(
        _msg_body,
        grid=(NB,),
        in_specs=[
            pl.BlockSpec((BE, NF), lambda i: (i, 0)),
            pl.BlockSpec((BE, NF), lambda i: (i, 0)),
            pl.BlockSpec((BE, 1), lambda i: (i, 0)),
            pl.BlockSpec((NF, 2 * NF), lambda i: (0, 0)),
            pl.BlockSpec((NF, 2 * NF), lambda i: (0, 0)),
            pl.BlockSpec((EF, 2 * NF), lambda i: (0, 0)),
            pl.BlockSpec((1, 2 * NF), lambda i: (0, 0)),
            pl.BlockSpec((1, 2 * NF), lambda i: (0, 0)),
            pl.BlockSpec((1, 2 * NF), lambda i: (0, 0)),
        ],
        out_specs=pl.BlockSpec((BE, NF), lambda i: (i, 0)),
        out_shape=jax.ShapeDtypeStruct((E, NF), jnp.float32),
    )(hs, hd, bond, ws, wd, we, bcat, scale, shift)


def _node_body(a0_ref, a1_ref, h_ref, g_ref, b_ref, out_ref):
    agg = a0_ref[0] + a1_ref[0]
    m = jnp.mean(agg, axis=0, keepdims=True)
    d = agg - m
    v = jnp.mean(d * d, axis=0, keepdims=True)
    bn = g_ref[...] * d * lax.rsqrt(v + 1e-5) + b_ref[...]
    y = h_ref[...] + bn
    out_ref[...] = jnp.maximum(y, 0.0) + jnp.log(1.0 + jnp.exp(-jnp.abs(y)))


def _node_update(aggp, h, g, b):
    return pl.pallas_call(
        _node_body,
        grid=(1,),
        in_specs=[
            pl.BlockSpec((1, N, NF), lambda i: (0, 0, 0)),
            pl.BlockSpec((1, N, NF), lambda i: (1, 0, 0)),
            pl.BlockSpec((N, NF), lambda i: (0, 0)),
            pl.BlockSpec((1, NF), lambda i: (0, 0)),
            pl.BlockSpec((1, NF), lambda i: (0, 0)),
        ],
        out_specs=pl.BlockSpec((N, NF), lambda i: (0, 0)),
        out_shape=jax.ShapeDtypeStruct((N, NF), jnp.float32),
    )(aggp, aggp, h, g, b)


def kernel(atom_features, bondlength, edge_index, W_emb, b_emb, Wi, bi, gi,
           bti, Wu, bu, gu, btu, gbn, bbn, W_fc, b_fc, W_out, b_out):
    src3 = edge_index[0].astype(jnp.int32).reshape(NW, NCHUNK, CH)
    dst3 = edge_index[1].astype(jnp.int32).reshape(NW, NCHUNK, CH)
    bond = bondlength.reshape(E, 1)
    zeros_n = jnp.zeros((N_PAD, NF), jnp.float32)

    h = atom_features @ W_emb + b_emb

    for l in range(L):
        ws = jnp.concatenate([Wi[l][:NF], Wu[l][:NF]], axis=1)
        wd = jnp.concatenate([Wi[l][NF:2 * NF], Wu[l][NF:2 * NF]], axis=1)
        we = jnp.concatenate([Wi[l][2 * NF:], Wu[l][2 * NF:]], axis=1)
        bcat = jnp.concatenate([bi[l], bu[l]]).reshape(1, 2 * NF)
        gcat = jnp.concatenate([gi[l], gu[l]])
        btcat = jnp.concatenate([bti[l], btu[l]])

        hs, hd = _sc_gather(h, src3, dst3)
        sums = _edge_stats(hs, hd, bond, ws, wd, we, bcat)
        mean = sums[0] / E
        var = sums[1] / E - mean * mean
        scale = (gcat * lax.rsqrt(var + 1e-5)).reshape(1, 2 * NF)
        shift = (btcat - mean * scale[0]).reshape(1, 2 * NF)
        msg = _edge_msg(hs, hd, bond, ws, wd, we, bcat, scale, shift)
        aggp = _sc_scatter(msg, dst3, zeros_n)
        h = _node_update(aggp, h, gbn[l].reshape(1, NF), bbn[l].reshape(1, NF))

    feat = jax.nn.softplus(h.mean(axis=0))
    feat = jax.nn.softplus(feat @ W_fc + b_fc)
    out = feat @ W_out + b_out
    return jnp.squeeze(out)


# speedup vs baseline: 1.0661x; 1.0109x over previous
import functools

import jax
import jax.numpy as jnp
from jax import lax
from jax.experimental import pallas as pl
from jax.experimental.pallas import tpu as pltpu
from jax.experimental.pallas import tpu_sc as plsc

N = 10000
E = 320000
NF = 128
EF = 32
L = 3

NC = 2
NS = 16
NW = NC * NS
EPW = E // NW
CH = 80
NCHUNK = EPW // CH
NPAIR = (NCHUNK - 1) // 2
RPS = 632
N_PAD = NS * RPS

BE = 10000
NB = E // BE


def _mesh():
    return plsc.VectorSubcoreMesh(
        core_axis_name="c", subcore_axis_name="s",
        num_cores=NC, num_subcores=NS)


@functools.cache
def _gather_kernel():
    @functools.partial(
        pl.kernel,
        out_type=(jax.ShapeDtypeStruct((E, NF), jnp.float32),
                  jax.ShapeDtypeStruct((E, NF), jnp.float32)),
        mesh=_mesh(),
        scratch_types=(pltpu.VMEM((NCHUNK, CH), jnp.int32),
                       pltpu.VMEM((NCHUNK, CH), jnp.int32),
                       pltpu.VMEM((CH, NF), jnp.float32),
                       pltpu.VMEM((CH, NF), jnp.float32),
                       pltpu.VMEM((CH, NF), jnp.float32),
                       pltpu.VMEM((CH, NF), jnp.float32),
                       pltpu.SemaphoreType.DMA,
                       pltpu.SemaphoreType.DMA,
                       pltpu.SemaphoreType.DMA,
                       pltpu.SemaphoreType.DMA),
    )
    def k(h_hbm, src3_hbm, dst3_hbm, hs_hbm, hd_hbm,
          sidx, didx, rs0, rd0, rs1, rd1, ss0, sd0, ss1, sd1):
        wid = lax.axis_index("s") * NC + lax.axis_index("c")
        pltpu.sync_copy(src3_hbm.at[wid], sidx)
        pltpu.sync_copy(dst3_hbm.at[wid], didx)

        def start(c, bs, bd, sems_, semd_):
            pltpu.async_copy(h_hbm.at[sidx.at[c]], bs, sems_)
            pltpu.async_copy(h_hbm.at[didx.at[c]], bd, semd_)

        def wait(bs, bd, sems_, semd_):
            pltpu.make_async_copy(h_hbm.at[sidx.at[0]], bs, sems_).wait()
            pltpu.make_async_copy(h_hbm.at[didx.at[0]], bd, semd_).wait()

        def drain(c, bs, bd):
            base = pl.multiple_of(wid * EPW + c * CH, 8)
            pltpu.sync_copy(bs, hs_hbm.at[pl.ds(base, CH)])
            pltpu.sync_copy(bd, hd_hbm.at[pl.ds(base, CH)])

        start(0, rs0, rd0, ss0, sd0)

        @pl.loop(0, NPAIR)
        def _pair(t):
            c0 = 2 * t
            start(c0 + 1, rs1, rd1, ss1, sd1)
            wait(rs0, rd0, ss0, sd0)
            drain(c0, rs0, rd0)
            start(c0 + 2, rs0, rd0, ss0, sd0)
            wait(rs1, rd1, ss1, sd1)
            drain(c0 + 1, rs1, rd1)

        wait(rs0, rd0, ss0, sd0)
        drain(NCHUNK - 1, rs0, rd0)

    return k


def _sc_gather(h, src3, dst3):
    return _gather_kernel()(h, src3, dst3)


@functools.cache
def _scatter_kernel(width):
    @functools.partial(
        pl.kernel,
        out_type=jax.ShapeDtypeStruct((NC, N_PAD, width), jnp.float32),
        mesh=_mesh(),
        scratch_types=(pltpu.VMEM_SHARED((N_PAD, width), jnp.float32),
                       pltpu.VMEM((CH, width), jnp.float32),
                       pltpu.VMEM((CH, width), jnp.float32),
                       pltpu.VMEM((NCHUNK, CH), jnp.int32),
                       pltpu.SemaphoreType.DMA,
                       pltpu.SemaphoreType.DMA),
    )
    def k(val_hbm, idx3_hbm, zeros_hbm, out_hbm, acc_sh, buf0, buf1, didx,
          sm0, sm1):
        cid = lax.axis_index("c")
        sid = lax.axis_index("s")
        wid = sid * NC + cid
        row0 = pl.multiple_of(sid * RPS, 8)
        pltpu.sync_copy(zeros_hbm.at[pl.ds(row0, RPS)],
                        acc_sh.at[pl.ds(row0, RPS)])
        pltpu.sync_copy(idx3_hbm.at[wid], didx)
        plsc.subcore_barrier()

        def start(c, buf, sem):
            base = pl.multiple_of(wid * EPW + c * CH, 8)
            pltpu.async_copy(val_hbm.at[pl.ds(base, CH)], buf, sem)

        def wait(buf, sem):
            pltpu.make_async_copy(val_hbm.at[pl.ds(0, CH)], buf, sem).wait()

        def scat(c, buf):
            pltpu.sync_copy(buf, acc_sh.at[didx.at[c]], add=True)

        start(0, buf0, sm0)

        @pl.loop(0, NPAIR)
        def _pair(t):
            c0 = 2 * t
            start(c0 + 1, buf1, sm1)
            wait(buf0, sm0)
            scat(c0, buf0)
            start(c0 + 2, buf0, sm0)
            wait(buf1, sm1)
            scat(c0 + 1, buf1)

        wait(buf0, sm0)
        scat(NCHUNK - 1, buf0)

        plsc.subcore_barrier()
        pltpu.sync_copy(acc_sh.at[pl.ds(row0, RPS)],
                        out_hbm.at[cid, pl.ds(row0, RPS)])

    return k


def _sc_scatter(vals, idx3, zeros):
    return _scatter_kernel(vals.shape[1])(vals, idx3, zeros)


def _rbf(bond_blk):
    centers = 1.0 + (4.0 / (EF - 1)) * lax.broadcasted_iota(
        jnp.int32, (1, EF), 1).astype(jnp.float32)
    gamma = (EF - 1) / 4.0
    return jnp.exp(-gamma * (bond_blk - centers) ** 2)


def _edge_x(hs, hd, bond, ws, wd, we, bcat):
    e = _rbf(bond)
    bf = jnp.bfloat16
    x = jnp.dot(hs.astype(bf), ws.astype(bf),
                preferred_element_type=jnp.float32)
    x += jnp.dot(hd.astype(bf), wd.astype(bf),
                 preferred_element_type=jnp.float32)
    x += jnp.dot(e.astype(bf), we.astype(bf),
                 preferred_element_type=jnp.float32)
    return x + bcat


def _stats_body(hs_ref, hd_ref, bond_ref, ws_ref, wd_ref, we_ref, bcat_ref,
                out_ref):
    i = pl.program_id(0)

    @pl.when(i == 0)
    def _():
        out_ref[...] = jnp.zeros_like(out_ref)

    x = _edge_x(hs_ref[...], hd_ref[...], bond_ref[...],
                ws_ref[...], wd_ref[...], we_ref[...], bcat_ref[...])
    out_ref[0:1, :] += jnp.sum(x, axis=0, keepdims=True)
    out_ref[1:2, :] += jnp.sum(x * x, axis=0, keepdims=True)


def _edge_stats(hs, hd, bond, ws, wd, we, bcat):
    return pl.pallas_call(
        _stats_body,
        grid=(NB,),
        in_specs=[
            pl.BlockSpec((BE, NF), lambda i: (i, 0)),
            pl.BlockSpec((BE, NF), lambda i: (i, 0)),
            pl.BlockSpec((BE, 1), lambda i: (i, 0)),
            pl.BlockSpec((NF, 2 * NF), lambda i: (0, 0)),
            pl.BlockSpec((NF, 2 * NF), lambda i: (0, 0)),
            pl.BlockSpec((EF, 2 * NF), lambda i: (0, 0)),
            pl.BlockSpec((1, 2 * NF), lambda i: (0, 0)),
        ],
        out_specs=pl.BlockSpec((8, 2 * NF), lambda i: (0, 0)),
        out_shape=jax.ShapeDtypeStruct((8, 2 * NF), jnp.float32),
    )(hs, hd, bond, ws, wd, we, bcat)


def _msg_body(hs_ref, hd_ref, bond_ref, ws_ref, wd_ref, we_ref, bcat_ref,
              scale_ref, shift_ref, out_ref):
    x = _edge_x(hs_ref[...], hd_ref[...], bond_ref[...],
                ws_ref[...], wd_ref[...], we_ref[...], bcat_ref[...])
    xh = x * scale_ref[...] + shift_ref[...]
    xi = xh[:, :NF]
    xu = xh[:, NF:]
    sig = 1.0 / (1.0 + jnp.exp(-xi))
    sp = jnp.maximum(xu, 0.0) + jnp.log(1.0 + jnp.exp(-jnp.abs(xu)))
    out_ref[...] = sig * sp


def _edge_msg(hs, hd, bond, ws, wd, we, bcat, scale, shift):
    return pl.pallas_call(
        _msg_body,
        grid=(NB,),
        in_specs=[
            pl.BlockSpec((BE, NF), lambda i: (i, 0)),
            pl.BlockSpec((BE, NF), lambda i: (i, 0)),
            pl.BlockSpec((BE, 1), lambda i: (i, 0)),
            pl.BlockSpec((NF, 2 * NF), lambda i: (0, 0)),
            pl.BlockSpec((NF, 2 * NF), lambda i: (0, 0)),
            pl.BlockSpec((EF, 2 * NF), lambda i: (0, 0)),
            pl.BlockSpec((1, 2 * NF), lambda i: (0, 0)),
            pl.BlockSpec((1, 2 * NF), lambda i: (0, 0)),
            pl.BlockSpec((1, 2 * NF), lambda i: (0, 0)),
        ],
        out_specs=pl.BlockSpec((BE, NF), lambda i: (i, 0)),
        out_shape=jax.ShapeDtypeStruct((E, NF), jnp.float32),
    )(hs, hd, bond, ws, wd, we, bcat, scale, shift)


def _node_body(a0_ref, a1_ref, h_ref, g_ref, b_ref, out_ref):
    agg = a0_ref[0] + a1_ref[0]
    m = jnp.mean(agg, axis=0, keepdims=True)
    d = agg - m
    v = jnp.mean(d * d, axis=0, keepdims=True)
    bn = g_ref[...] * d * lax.rsqrt(v + 1e-5) + b_ref[...]
    y = h_ref[...] + bn
    out_ref[...] = jnp.maximum(y, 0.0) + jnp.log(1.0 + jnp.exp(-jnp.abs(y)))


def _node_update(aggp, h, g, b):
    return pl.pallas_call(
        _node_body,
        grid=(1,),
        in_specs=[
            pl.BlockSpec((1, N, NF), lambda i: (0, 0, 0)),
            pl.BlockSpec((1, N, NF), lambda i: (1, 0, 0)),
            pl.BlockSpec((N, NF), lambda i: (0, 0)),
            pl.BlockSpec((1, NF), lambda i: (0, 0)),
            pl.BlockSpec((1, NF), lambda i: (0, 0)),
        ],
        out_specs=pl.BlockSpec((N, NF), lambda i: (0, 0)),
        out_shape=jax.ShapeDtypeStruct((N, NF), jnp.float32),
    )(aggp, aggp, h, g, b)


def kernel(atom_features, bondlength, edge_index, W_emb, b_emb, Wi, bi, gi,
           bti, Wu, bu, gu, btu, gbn, bbn, W_fc, b_fc, W_out, b_out):
    src3 = edge_index[0].astype(jnp.int32).reshape(NW, NCHUNK, CH)
    dst3 = edge_index[1].astype(jnp.int32).reshape(NW, NCHUNK, CH)
    bond = bondlength.reshape(E, 1)
    zeros_n = jnp.zeros((N_PAD, NF), jnp.float32)

    h = atom_features @ W_emb + b_emb

    for l in range(L):
        ws = jnp.concatenate([Wi[l][:NF], Wu[l][:NF]], axis=1)
        wd = jnp.concatenate([Wi[l][NF:2 * NF], Wu[l][NF:2 * NF]], axis=1)
        we = jnp.concatenate([Wi[l][2 * NF:], Wu[l][2 * NF:]], axis=1)
        bcat = jnp.concatenate([bi[l], bu[l]]).reshape(1, 2 * NF)
        gcat = jnp.concatenate([gi[l], gu[l]])
        btcat = jnp.concatenate([bti[l], btu[l]])

        hs, hd = _sc_gather(h, src3, dst3)
        sums = _edge_stats(hs, hd, bond, ws, wd, we, bcat)
        mean = sums[0] / E
        var = sums[1] / E - mean * mean
        scale = (gcat * lax.rsqrt(var + 1e-5)).reshape(1, 2 * NF)
        shift = (btcat - mean * scale[0]).reshape(1, 2 * NF)
        msg = _edge_msg(hs, hd, bond, ws, wd, we, bcat, scale, shift)
        aggp = _sc_scatter(msg, dst3, zeros_n)
        h = _node_update(aggp, h, gbn[l].reshape(1, NF), bbn[l].reshape(1, NF))

    feat = jax.nn.softplus(h.mean(axis=0))
    feat = jax.nn.softplus(feat @ W_fc + b_fc)
    out = feat @ W_out + b_out
    return jnp.squeeze(out)
